# baseline (speedup 1.0000x reference)
import functools

import jax
import jax.numpy as jnp
from jax import lax
from jax.experimental import pallas as pl
from jax.experimental.pallas import tpu as pltpu
from jax.experimental.pallas import tpu_sc as plsc

B = 8
N = 8192
G = 512
M = 32
GT = 256
TW = 16


def _fps_body(xyz_ref, cent_ref, dists_ref):
    coli = lax.broadcasted_iota(jnp.int32, (B, N), 1)
    gcol = lax.broadcasted_iota(jnp.int32, (1, G), 1)
    dists_ref[:] = jnp.full((B, N), 1e10, jnp.float32)
    cent_ref[:] = jnp.zeros((3, B, G), jnp.float32)

    def body(g, far):
        x = xyz_ref[0]
        y = xyz_ref[1]
        z = xyz_ref[2]
        mask = coli == far
        cx = jnp.sum(jnp.where(mask, x, 0.0), axis=1, keepdims=True)
        cy = jnp.sum(jnp.where(mask, y, 0.0), axis=1, keepdims=True)
        cz = jnp.sum(jnp.where(mask, z, 0.0), axis=1, keepdims=True)
        oh = gcol == g
        cent_ref[0] = cent_ref[0] + jnp.where(oh, cx, 0.0)
        cent_ref[1] = cent_ref[1] + jnp.where(oh, cy, 0.0)
        cent_ref[2] = cent_ref[2] + jnp.where(oh, cz, 0.0)
        d = (x - cx) ** 2 + (y - cy) ** 2 + (z - cz) ** 2
        nd = jnp.minimum(dists_ref[:], d)
        dists_ref[:] = nd
        mx = jnp.max(nd, axis=1, keepdims=True)
        far2 = jnp.min(jnp.where(nd == mx, coli, N), axis=1, keepdims=True)
        return far2

    lax.fori_loop(0, G, body, jnp.zeros((B, 1), jnp.int32))


def _knn_body(xyz_ref, cent_ref, idx_ref, d2_ref, b=0):
    xr = xyz_ref[0]
    cr = cent_ref[0]
    x = xr[0:1, :]
    y = xr[1:2, :]
    z = xr[2:3, :]
    cx = cr[0:1, :].reshape(GT, 1)
    cy = cr[1:2, :].reshape(GT, 1)
    cz = cr[2:3, :].reshape(GT, 1)
    d2_ref[:] = (cx - x) ** 2 + (cy - y) ** 2 + (cz - z) ** 2
    coli = lax.broadcasted_iota(jnp.int32, (GT, N), 1)
    mcol = lax.broadcasted_iota(jnp.int32, (1, M), 1)
    base = b * N

    def body(m, acc):
        d = d2_ref[:]
        mn = jnp.min(d, axis=1, keepdims=True)
        am = jnp.min(jnp.where(d == mn, coli, N), axis=1, keepdims=True)
        acc = acc + jnp.where(mcol == m, am + base, 0)
        d2_ref[:] = jnp.where(coli == am, jnp.inf, d)
        return acc

    idx_ref[0] = lax.fori_loop(0, M, body, jnp.zeros((GT, M), jnp.int32))


def _fps_call(xyzT):
    return pl.pallas_call(
        _fps_body,
        out_shape=jax.ShapeDtypeStruct((3, B, G), jnp.float32),
        in_specs=[pl.BlockSpec((3, B, N), lambda: (0, 0, 0))],
        out_specs=pl.BlockSpec((3, B, G), lambda: (0, 0, 0)),
        scratch_shapes=[pltpu.VMEM((B, N), jnp.float32)],
    )(xyzT)


def _knn_call(xyzB, centersB, b):
    return pl.pallas_call(
        functools.partial(_knn_body, b=b),
        grid=(G // GT,),
        out_shape=jax.ShapeDtypeStruct((1, G, M), jnp.int32),
        in_specs=[
            pl.BlockSpec((1, 3, N), lambda t: (b, 0, 0)),
            pl.BlockSpec((1, 3, GT), lambda t: (b, 0, t)),
        ],
        out_specs=pl.BlockSpec((1, GT, M), lambda t: (0, t, 0)),
        scratch_shapes=[pltpu.VMEM((GT, N), jnp.float32)],
    )(xyzB, centersB)


NW = 32
PPW = (G * M) // NW
GPW = PPW // M


def _sc_gather_body(tab_hbm, cent_hbm, idx_hbm, out_hbm,
                    idxv, rows_v, cent_v, sem):
    cid = lax.axis_index("c")
    sid = lax.axis_index("s")
    wid = sid * 2 + cid
    pbase = wid * PPW
    gbase = wid * GPW
    pltpu.sync_copy(idx_hbm.at[pl.ds(pbase, PPW)], idxv)
    pltpu.sync_copy(cent_hbm.at[pl.ds(gbase, GPW)], cent_v)
    pltpu.async_copy(tab_hbm.at[idxv], rows_v, sem).wait()

    def body(i, _):
        c = cent_v[i]
        r = i * M
        for j in range(M):
            rows_v[r + j] = rows_v[r + j] - c
        return 0

    lax.fori_loop(0, GPW, body, 0)

    pltpu.sync_copy(rows_v, out_hbm.at[pl.ds(pbase, PPW)])


def _sc_gather_call(tab, cent, idxf):
    mesh = plsc.VectorSubcoreMesh(core_axis_name="c", subcore_axis_name="s")
    f32 = jnp.float32
    run = pl.kernel(
        _sc_gather_body,
        mesh=mesh,
        out_type=jax.ShapeDtypeStruct((G * M, TW), f32),
        scratch_types=[
            pltpu.VMEM((PPW,), jnp.int32),
            pltpu.VMEM((PPW, TW), f32),
            pltpu.VMEM((GPW, 16), f32),
            pltpu.SemaphoreType.DMA,
        ],
        compiler_params=pltpu.CompilerParams(use_tc_tiling_on_sc=False),
    )
    return run(tab, cent, idxf)


def kernel(xyz):
    xyzT = jnp.transpose(xyz, (2, 0, 1))
    centers = _fps_call(xyzT)
    xyzB = jnp.transpose(xyz, (0, 2, 1))
    centersB = jnp.transpose(centers, (1, 0, 2))
    center = jnp.transpose(centers, (1, 2, 0))
    tab = jnp.pad(xyz.reshape(B * N, 3), ((0, 0), (0, TW - 3)))
    cent = jnp.pad(center.reshape(B * G, 3), ((0, 0), (0, TW - 3)))
    outs = []
    for b in range(B):
        idx_b = _knn_call(xyzB, centersB, b)
        outs.append(_sc_gather_call(tab, cent[b * G:(b + 1) * G], idx_b.reshape(-1)))
    out = jnp.concatenate(outs, axis=0)
    neighborhood = out[:, :3].reshape(B, G, M, 3)
    return neighborhood, center


# speedup vs baseline: 1.0296x; 1.0296x over previous
import jax
import jax.numpy as jnp
from jax import lax
from jax.experimental import pallas as pl
from jax.experimental.pallas import tpu as pltpu
from jax.experimental.pallas import tpu_sc as plsc

B = 8
N = 8192
G = 512
M = 32
GT = 256
TW = 16


def _fps_body(xyz_ref, cent_ref, dists_ref):
    coli = lax.broadcasted_iota(jnp.int32, (B, N), 1)
    gcol = lax.broadcasted_iota(jnp.int32, (1, G), 1)
    dists_ref[:] = jnp.full((B, N), 1e10, jnp.float32)
    cent_ref[:] = jnp.zeros((3, B, G), jnp.float32)

    def body(g, far):
        x = xyz_ref[0]
        y = xyz_ref[1]
        z = xyz_ref[2]
        mask = coli == far
        cx = jnp.sum(jnp.where(mask, x, 0.0), axis=1, keepdims=True)
        cy = jnp.sum(jnp.where(mask, y, 0.0), axis=1, keepdims=True)
        cz = jnp.sum(jnp.where(mask, z, 0.0), axis=1, keepdims=True)
        oh = gcol == g
        cent_ref[0] = cent_ref[0] + jnp.where(oh, cx, 0.0)
        cent_ref[1] = cent_ref[1] + jnp.where(oh, cy, 0.0)
        cent_ref[2] = cent_ref[2] + jnp.where(oh, cz, 0.0)
        d = (x - cx) ** 2 + (y - cy) ** 2 + (z - cz) ** 2
        nd = jnp.minimum(dists_ref[:], d)
        dists_ref[:] = nd
        mx = jnp.max(nd, axis=1, keepdims=True)
        far2 = jnp.min(jnp.where(nd == mx, coli, N), axis=1, keepdims=True)
        return far2

    lax.fori_loop(0, G, body, jnp.zeros((B, 1), jnp.int32))


def _knn_body(xyz_ref, cent_ref, idx_ref, d2_ref):
    xr = xyz_ref[0]
    cr = cent_ref[0]
    x = xr[0:1, :]
    y = xr[1:2, :]
    z = xr[2:3, :]
    cx = cr[0:1, :].reshape(GT, 1)
    cy = cr[1:2, :].reshape(GT, 1)
    cz = cr[2:3, :].reshape(GT, 1)
    d2_ref[:] = (cx - x) ** 2 + (cy - y) ** 2 + (cz - z) ** 2
    coli = lax.broadcasted_iota(jnp.int32, (GT, N), 1)
    mcol = lax.broadcasted_iota(jnp.int32, (1, M), 1)
    base = pl.program_id(0) * N

    def body(m, acc):
        d = d2_ref[:]
        mn = jnp.min(d, axis=1, keepdims=True)
        am = jnp.min(jnp.where(d == mn, coli, N), axis=1, keepdims=True)
        acc = acc + jnp.where(mcol == m, am + base, 0)
        d2_ref[:] = jnp.where(coli == am, jnp.inf, d)
        return acc

    idx_ref[0] = lax.fori_loop(0, M, body, jnp.zeros((GT, M), jnp.int32))


def _fps_call(xyzT):
    return pl.pallas_call(
        _fps_body,
        out_shape=jax.ShapeDtypeStruct((3, B, G), jnp.float32),
        in_specs=[pl.BlockSpec((3, B, N), lambda: (0, 0, 0))],
        out_specs=pl.BlockSpec((3, B, G), lambda: (0, 0, 0)),
        scratch_shapes=[pltpu.VMEM((B, N), jnp.float32)],
    )(xyzT)


def _knn_call(xyzB, centersB):
    return pl.pallas_call(
        _knn_body,
        grid=(B, G // GT),
        out_shape=jax.ShapeDtypeStruct((B, G, M), jnp.int32),
        in_specs=[
            pl.BlockSpec((1, 3, N), lambda b, t: (b, 0, 0)),
            pl.BlockSpec((1, 3, GT), lambda b, t: (b, 0, t)),
        ],
        out_specs=pl.BlockSpec((1, GT, M), lambda b, t: (b, t, 0)),
        scratch_shapes=[pltpu.VMEM((GT, N), jnp.float32)],
    )(xyzB, centersB)


NW = 32
PPW = (B * G * M) // NW
GPW = PPW // M


def _sc_gather_body(tab_hbm, cent_hbm, idx_hbm, out_hbm,
                    idxv, rows_v, cent_v, sem):
    cid = lax.axis_index("c")
    sid = lax.axis_index("s")
    wid = sid * 2 + cid
    pbase = wid * PPW
    gbase = wid * GPW
    pltpu.sync_copy(idx_hbm.at[pl.ds(pbase, PPW)], idxv)
    pltpu.sync_copy(cent_hbm.at[pl.ds(gbase, GPW)], cent_v)
    pltpu.async_copy(tab_hbm.at[idxv], rows_v, sem).wait()

    def body(i, _):
        c = cent_v[i]
        r = i * M
        for j in range(M):
            rows_v[r + j] = rows_v[r + j] - c
        return 0

    lax.fori_loop(0, GPW, body, 0)

    pltpu.sync_copy(rows_v, out_hbm.at[pl.ds(pbase, PPW)])


def _sc_gather_call(tab, cent, idxf):
    mesh = plsc.VectorSubcoreMesh(core_axis_name="c", subcore_axis_name="s")
    f32 = jnp.float32
    run = pl.kernel(
        _sc_gather_body,
        mesh=mesh,
        out_type=jax.ShapeDtypeStruct((B * G * M, TW), f32),
        scratch_types=[
            pltpu.VMEM((PPW,), jnp.int32),
            pltpu.VMEM((PPW, TW), f32),
            pltpu.VMEM((GPW, 16), f32),
            pltpu.SemaphoreType.DMA,
        ],
        compiler_params=pltpu.CompilerParams(use_tc_tiling_on_sc=False),
    )
    return run(tab, cent, idxf)


def kernel(xyz):
    xyzT = jnp.transpose(xyz, (2, 0, 1))
    centers = _fps_call(xyzT)
    xyzB = jnp.transpose(xyz, (0, 2, 1))
    centersB = jnp.transpose(centers, (1, 0, 2))
    idx = _knn_call(xyzB, centersB)
    center = jnp.transpose(centers, (1, 2, 0))
    tab = jnp.pad(xyz.reshape(B * N, 3), ((0, 0), (0, TW - 3)))
    cent = jnp.pad(center.reshape(B * G, 3), ((0, 0), (0, TW - 3)))
    out = _sc_gather_call(tab, cent, idx.reshape(-1))
    neighborhood = out[:, :3].reshape(B, G, M, 3)
    return neighborhood, center
